# Initial kernel scaffold; baseline (speedup 1.0000x reference)
import jax
import jax.numpy as jnp
from jax.experimental import pallas as pl


def kernel(hidden_states, gate_weight, e_score_correction_bias, expert_gate, expert_up, expert_down, shared_gate, shared_up, shared_down):
    raise NotImplementedError("write your pallas kernel here")


# speedup vs baseline: 1.8683x; 1.8683x over previous
import functools

import jax
import jax.numpy as jnp
from jax.experimental import pallas as pl
from jax.experimental.pallas import tpu as pltpu

TOP_K = 2
ROUTED_SCALING_FACTOR = 2.5


def _router_body(x_ref, gw_ref, bias_ref, combine_ref):
    x = x_ref[...]
    gw = gw_ref[...]
    logits = jax.lax.dot_general(x, gw, (((1,), (1,)), ((), ())),
                                 preferred_element_type=jnp.float32)
    scores = jax.nn.sigmoid(logits)
    s = scores + bias_ref[...]
    t, e = s.shape
    iota = jax.lax.broadcasted_iota(jnp.int32, (t, e), 1)
    big = jnp.asarray(e, jnp.int32)
    m1 = jnp.max(s, axis=1, keepdims=True)
    i1 = jnp.min(jnp.where(s == m1, iota, big), axis=1, keepdims=True)
    oh1 = (iota == i1)
    s2 = jnp.where(oh1, -jnp.inf, s)
    m2 = jnp.max(s2, axis=1, keepdims=True)
    i2 = jnp.min(jnp.where(s2 == m2, iota, big), axis=1, keepdims=True)
    oh2 = (iota == i2)
    w1 = jnp.sum(jnp.where(oh1, scores, 0.0), axis=1, keepdims=True)
    w2 = jnp.sum(jnp.where(oh2, scores, 0.0), axis=1, keepdims=True)
    scale = ROUTED_SCALING_FACTOR / (w1 + w2 + 1e-20)
    combine_ref[...] = (jnp.where(oh1, w1, 0.0) + jnp.where(oh2, w2, 0.0)) * scale


def _router(x, gate_weight, bias):
    t, d = x.shape
    e = gate_weight.shape[0]
    return pl.pallas_call(
        _router_body,
        out_shape=jax.ShapeDtypeStruct((t, e), jnp.float32),
    )(x, gate_weight, bias.reshape(1, e))


def _moe_dense_body(x_ref, comb_ref, wg_ref, wu_ref, wd_ref, out_ref):
    e = pl.program_id(0)
    f = pl.program_id(1)

    @pl.when(jnp.logical_and(e == 0, f == 0))
    def _():
        out_ref[...] = jnp.zeros_like(out_ref)

    x = x_ref[...]
    g = jax.lax.dot_general(x, wg_ref[0], (((1,), (1,)), ((), ())),
                            preferred_element_type=jnp.float32)
    u = jax.lax.dot_general(x, wu_ref[0], (((1,), (1,)), ((), ())),
                            preferred_element_type=jnp.float32)
    h = (g * jax.nn.sigmoid(g)) * u
    o = jax.lax.dot_general(h, wd_ref[0], (((1,), (1,)), ((), ())),
                            preferred_element_type=jnp.float32)
    comb = comb_ref[...]
    lane = jax.lax.broadcasted_iota(jnp.int32, comb.shape, 1)
    wcol = jnp.sum(jnp.where(lane == e, comb, 0.0), axis=1, keepdims=True)
    out_ref[...] += o * wcol


def _moe_dense(x, combine, expert_gate, expert_up, expert_down, ffc=256):
    t, d = x.shape
    ne, dff, _ = expert_gate.shape
    ffc = min(ffc, dff)
    nf = dff // ffc
    return pl.pallas_call(
        _moe_dense_body,
        grid=(ne, nf),
        in_specs=[
            pl.BlockSpec((t, d), lambda e, f: (0, 0)),
            pl.BlockSpec((t, ne), lambda e, f: (0, 0)),
            pl.BlockSpec((1, ffc, d), lambda e, f: (e, f, 0)),
            pl.BlockSpec((1, ffc, d), lambda e, f: (e, f, 0)),
            pl.BlockSpec((1, d, ffc), lambda e, f: (e, 0, f)),
        ],
        out_specs=pl.BlockSpec((t, d), lambda e, f: (0, 0)),
        out_shape=jax.ShapeDtypeStruct((t, d), jnp.float32),
    )(x, combine, expert_gate, expert_up, expert_down)


def _shared_body(x_ref, wg_ref, wu_ref, wd_ref, out_ref):
    f = pl.program_id(0)

    @pl.when(f == 0)
    def _():
        out_ref[...] = jnp.zeros_like(out_ref)

    x = x_ref[...]
    g = jax.lax.dot_general(x, wg_ref[...], (((1,), (1,)), ((), ())),
                            preferred_element_type=jnp.float32)
    u = jax.lax.dot_general(x, wu_ref[...], (((1,), (1,)), ((), ())),
                            preferred_element_type=jnp.float32)
    h = (g * jax.nn.sigmoid(g)) * u
    out_ref[...] += jax.lax.dot_general(h, wd_ref[...], (((1,), (1,)), ((), ())),
                                        preferred_element_type=jnp.float32)


def _shared_mlp(x, wg, wu, wd, ffc=256):
    t, d = x.shape
    dffs = wg.shape[0]
    ffc = min(ffc, dffs)
    nf = dffs // ffc
    return pl.pallas_call(
        _shared_body,
        grid=(nf,),
        in_specs=[
            pl.BlockSpec((t, d), lambda f: (0, 0)),
            pl.BlockSpec((ffc, d), lambda f: (f, 0)),
            pl.BlockSpec((ffc, d), lambda f: (f, 0)),
            pl.BlockSpec((d, ffc), lambda f: (0, f)),
        ],
        out_specs=pl.BlockSpec((t, d), lambda f: (0, 0)),
        out_shape=jax.ShapeDtypeStruct((t, d), jnp.float32),
    )(x, wg, wu, wd)


def kernel(hidden_states, gate_weight, e_score_correction_bias, expert_gate,
           expert_up, expert_down, shared_gate, shared_up, shared_down):
    b, s, d = hidden_states.shape
    x = hidden_states.reshape(-1, d)
    combine = _router(x, gate_weight, e_score_correction_bias)
    routed = _moe_dense(x, combine, expert_gate, expert_up, expert_down)
    shared = _shared_mlp(x, shared_gate, shared_up, shared_down)
    return (routed + shared).reshape(hidden_states.shape)
